# baseline (speedup 1.0000x reference)
import functools

import jax
import jax.numpy as jnp
from jax import lax
from jax.experimental import pallas as pl
from jax.experimental.pallas import tpu as pltpu, tpu_sc as plsc

_M = 4096
_N = 12000
_NW = 32
_RPW = _M // _NW
_CH = 4
_NCHUNK = _RPW // _CH


def _make_sc_copy():
    mesh = plsc.VectorSubcoreMesh(core_axis_name="c", subcore_axis_name="s")

    @functools.partial(
        pl.kernel, mesh=mesh,
        out_type=jax.ShapeDtypeStruct((_M, _N), jnp.float32),
        scratch_types=[
            pltpu.VMEM((_CH, _N), jnp.float32),
            pltpu.VMEM((_CH, _N), jnp.float32),
            pltpu.SemaphoreType.DMA,
            pltpu.SemaphoreType.DMA,
            pltpu.SemaphoreType.DMA,
            pltpu.SemaphoreType.DMA,
        ],
    )
    def k(src_hbm, out_hbm, buf0, buf1, rs0, rs1, ws0, ws1):
        wid = lax.axis_index("s") * 2 + lax.axis_index("c")
        base = wid * _RPW
        bufs = (buf0, buf1)
        rsems = (rs0, rs1)
        wsems = (ws0, ws1)
        pltpu.async_copy(src_hbm.at[pl.ds(base, _CH)], buf0, rs0)
        pltpu.async_copy(src_hbm.at[pl.ds(base + _CH, _CH)], buf1, rs1)
        for i in range(_NCHUNK):
            s = i % 2
            r = base + i * _CH
            pltpu.make_async_copy(
                src_hbm.at[pl.ds(r, _CH)], bufs[s], rsems[s]).wait()
            if i >= 2:
                pass
            pltpu.async_copy(bufs[s], out_hbm.at[pl.ds(r, _CH)], wsems[s])
            if i + 2 < _NCHUNK:
                pltpu.make_async_copy(
                    bufs[s], out_hbm.at[pl.ds(r, _CH)], wsems[s]).wait()
                pltpu.async_copy(
                    src_hbm.at[pl.ds(base + (i + 2) * _CH, _CH)],
                    bufs[s], rsems[s])
            else:
                pltpu.make_async_copy(
                    bufs[s], out_hbm.at[pl.ds(r, _CH)], wsems[s]).wait()

    return k


_sc_copy = _make_sc_copy()


@jax.jit
def kernel(x, emb_ck, emb_fc, emb_do, emb_bs, emb_lr, emb_mo,
           W1, b1, W2, b2, W3, b3):
    src = jnp.broadcast_to(b3.reshape(1, _N), (_M, _N)) + 1.0
    return _sc_copy(src)


# speedup vs baseline: 1.5698x; 1.5698x over previous
import functools

import jax
import jax.numpy as jnp
from jax.experimental import pallas as pl
from jax.experimental.pallas import tpu as pltpu

_M = 4096
_H1 = 128
_H2 = 64
_N = 12000
_BM = 256

_VOCABS = (3, 4, 5, 4, 10, 5)


def _fused_kernel(x_ref, ck_ref, fc_ref, do_ref, bs_ref, lr_ref, mo_ref,
                  w1_ref, b1_ref, w2_ref, b2_ref, w3_ref, b3_ref,
                  out_ref, w3b_scr):
    i = pl.program_id(0)

    @pl.when(i == 0)
    def _cast_w3():
        w3b_scr[:] = w3_ref[:].astype(jnp.bfloat16)

    x = x_ref[:]
    acc = jnp.broadcast_to(b1_ref[:], (_BM, _H1))
    tables = (ck_ref, fc_ref, do_ref, bs_ref, lr_ref, mo_ref)
    for j in range(6):
        voc = _VOCABS[j]
        col = jax.lax.slice(x, (0, j), (_BM, j + 1))
        oh = (col == jax.lax.broadcasted_iota(
            jnp.int32, (_BM, voc), 1)).astype(jnp.float32)
        tj = jnp.dot(tables[j][:], w1_ref[4 * j:4 * j + 4, :],
                     preferred_element_type=jnp.float32)
        acc = acc + jnp.dot(oh, tj, preferred_element_type=jnp.float32)
    h1 = jnp.maximum(acc, 0.0)
    h2 = jnp.dot(h1, w2_ref[:], preferred_element_type=jnp.float32)
    h2 = jnp.maximum(h2 + b2_ref[:], 0.0)
    out_ref[:] = (
        jnp.dot(h2.astype(jnp.bfloat16), w3b_scr[:],
                preferred_element_type=jnp.float32)
        + b3_ref[:]
    )


@jax.jit
def kernel(x, emb_ck, emb_fc, emb_do, emb_bs, emb_lr, emb_mo,
           W1, b1, W2, b2, W3, b3):
    grid = (_M // _BM,)
    full = lambda shape: pl.BlockSpec(shape, lambda i: (0,) * len(shape))
    out = pl.pallas_call(
        _fused_kernel,
        grid=grid,
        in_specs=[
            pl.BlockSpec((_BM, 6), lambda i: (i, 0)),
            full((3, 4)), full((4, 4)), full((5, 4)),
            full((4, 4)), full((10, 4)), full((5, 4)),
            full((24, _H1)), full((1, _H1)),
            full((_H1, _H2)), full((1, _H2)),
            full((_H2, _N)),
            full((1, _N)),
        ],
        out_specs=pl.BlockSpec((_BM, _N), lambda i: (i, 0)),
        out_shape=jax.ShapeDtypeStruct((_M, _N), jnp.float32),
        scratch_shapes=[pltpu.VMEM((_H2, _N), jnp.bfloat16)],
        compiler_params=pltpu.CompilerParams(
            dimension_semantics=("arbitrary",),
        ),
    )(x, emb_ck, emb_fc, emb_do, emb_bs, emb_lr, emb_mo,
      W1, b1.reshape(1, _H1), W2, b2.reshape(1, _H2), W3,
      b3.reshape(1, _N))
    return out
